# Initial kernel scaffold; baseline (speedup 1.0000x reference)
import jax
import jax.numpy as jnp
from jax.experimental import pallas as pl


def kernel(x, emb_table):
    raise NotImplementedError("write your pallas kernel here")


# speedup vs baseline: 1.6305x; 1.6305x over previous
import jax
import jax.numpy as jnp
from jax.experimental import pallas as pl


def _add_kernel(x_ref, emb_ref, o_ref):
    o_ref[...] = x_ref[...] + emb_ref[...][None, :, :]


def _pos_add_3d(x, emb_slice):
    B, L, D = x.shape
    BLK = 256
    grid = (L // BLK,)
    return pl.pallas_call(
        _add_kernel,
        grid=grid,
        in_specs=[
            pl.BlockSpec((B, BLK, D), lambda i: (0, i, 0)),
            pl.BlockSpec((BLK, D), lambda i: (i, 0)),
        ],
        out_specs=pl.BlockSpec((B, BLK, D), lambda i: (0, i, 0)),
        out_shape=jax.ShapeDtypeStruct((B, L, D), x.dtype),
    )(x, emb_slice)


def kernel(x, emb_table):
    if x.ndim == 3:
        L = x.shape[-2]
        return _pos_add_3d(x, emb_table[:L])
    b, h, l, d = x.shape
    xr = jnp.reshape(jnp.transpose(x, (0, 2, 1, 3)), (b, l, h * d))
    xr = _pos_add_3d(xr, emb_table[:l])
    return jnp.transpose(jnp.reshape(xr, (b, l, h, d)), (0, 2, 1, 3))
